# baseline (speedup 1.0000x reference)
import functools

import jax
import jax.numpy as jnp
from jax import lax
from jax.experimental import pallas as pl
from jax.experimental.pallas import tpu as pltpu
from jax.experimental.pallas import tpu_sc as plsc

N = 10000
D = 128
DH = D // 2
E = 320000

NC = 2
NS = 16
C = 128
ET = 20480
K = ET // C
EP = NS * ET
NP = 10240
RPT = NP // NS


NBUF = 4


def _sc_agg_body(with_cnt, *refs):
    if with_cnt:
        (xs_hbm, src_hbm, dst_hbm, acc_out, cnt_out,
         src_v, dst_v, rows, ones16, zeros16, acc_sh, cnt_sh,
         gsem, ssem, csem) = refs
    else:
        (xs_hbm, src_hbm, dst_hbm, acc_out,
         src_v, dst_v, rows, acc_sh, gsem, ssem) = refs
        csem = None

    c = lax.axis_index("c")
    s = lax.axis_index("s")

    pltpu.sync_copy(src_hbm.at[s], src_v)
    pltpu.sync_copy(dst_hbm.at[s], dst_v)

    zv = jnp.zeros((16,), jnp.float32)

    def zrow(r, carry):
        for q in range(DH // 16):
            rows[0, r, pl.ds(16 * q, 16)] = zv
        return carry

    lax.fori_loop(0, C, zrow, 0)
    for t in range(RPT // C):
        pltpu.sync_copy(rows.at[0], acc_sh.at[pl.ds(s * RPT + t * C, C)])

    if with_cnt:
        ov = jnp.full((16,), 1.0, jnp.float32)

        def orow(r, carry):
            ones16[r, pl.ds(0, 16)] = ov
            zeros16[r, pl.ds(0, 16)] = zv
            return carry

        lax.fori_loop(0, C, orow, 0)
        for t in range(RPT // C):
            pltpu.sync_copy(zeros16, cnt_sh.at[pl.ds(s * RPT + t * C, C)])

    plsc.subcore_barrier()

    xh = xs_hbm.at[c]
    for b in range(NBUF):
        pltpu.async_copy(xh.at[src_v.at[b]], rows.at[b], gsem.at[b])

    def step(t, carry):
        j0 = NBUF * t
        want_cnt = jnp.where(c == 0, j0 < K // 2, j0 >= K // 2)
        for b in range(NBUF):
            j = j0 + b
            pltpu.make_async_copy(xh.at[src_v.at[j]], rows.at[b], gsem.at[b]).wait()
            pltpu.async_copy(rows.at[b], acc_sh.at[dst_v.at[j]], ssem.at[b], add=True)
            if with_cnt:
                @pl.when(want_cnt)
                def _():
                    pltpu.async_copy(ones16, cnt_sh.at[dst_v.at[j]], csem,
                                     add=True)

        @pl.when(t < K // NBUF - 1)
        def _():
            for b in range(NBUF):
                j = j0 + b
                pltpu.make_async_copy(rows.at[b], acc_sh.at[dst_v.at[j]],
                                      ssem.at[b]).wait()
                pltpu.async_copy(xh.at[src_v.at[j + NBUF]], rows.at[b],
                                 gsem.at[b])
            if with_cnt:
                @pl.when(want_cnt)
                def _():
                    for b in range(NBUF):
                        pltpu.make_async_copy(ones16,
                                              cnt_sh.at[dst_v.at[j0 + b]],
                                              csem).wait()

        return carry

    lax.fori_loop(0, K // NBUF, step, 0)

    for b in range(NBUF):
        j = K - NBUF + b
        pltpu.make_async_copy(rows.at[b], acc_sh.at[dst_v.at[j]], ssem.at[b]).wait()
        if with_cnt:
            @pl.when(c != 0)
            def _():
                pltpu.make_async_copy(ones16, cnt_sh.at[dst_v.at[j]],
                                      csem).wait()

    plsc.subcore_barrier()

    pltpu.sync_copy(acc_sh.at[pl.ds(s * RPT, RPT)],
                    acc_out.at[c, pl.ds(s * RPT, RPT)])
    if with_cnt:
        pltpu.sync_copy(cnt_sh.at[pl.ds(s * RPT, RPT)],
                        cnt_out.at[c, pl.ds(s * RPT, RPT)])


def _make_sc_agg(with_cnt):
    mesh = plsc.VectorSubcoreMesh(core_axis_name="c", subcore_axis_name="s")
    out_type = [jax.ShapeDtypeStruct((NC, NP, DH), jnp.float32)]
    scratch = [
        pltpu.VMEM((K, C), jnp.int32),
        pltpu.VMEM((K, C), jnp.int32),
        pltpu.VMEM((NBUF, C, DH), jnp.float32),
    ]
    if with_cnt:
        out_type.append(jax.ShapeDtypeStruct((NC, NP, 16), jnp.float32))
        scratch += [
            pltpu.VMEM((C, 16), jnp.float32),
            pltpu.VMEM((C, 16), jnp.float32),
        ]
    scratch.append(pltpu.VMEM_SHARED((NP, DH), jnp.float32))
    if with_cnt:
        scratch.append(pltpu.VMEM_SHARED((NP, 16), jnp.float32))
    scratch += [pltpu.SemaphoreType.DMA((NBUF,)),
                pltpu.SemaphoreType.DMA((NBUF,))]
    if with_cnt:
        scratch.append(pltpu.SemaphoreType.DMA)

    return pl.kernel(
        functools.partial(_sc_agg_body, with_cnt),
        out_type=tuple(out_type),
        mesh=mesh,
        scratch_types=tuple(scratch),
        compiler_params=pltpu.CompilerParams(use_tc_tiling_on_sc=False),
    )


def _combine_body(relu, pa_ref, pc_ref, xin_ref, wl_ref, wr_ref, b_ref,
                  out_ref):
    acc = jnp.concatenate([pa_ref[0], pa_ref[1]], axis=1)
    xin = jnp.concatenate([xin_ref[0], xin_ref[1]], axis=1)
    cnt = pc_ref[0, :, 0:1] + pc_ref[1, :, 0:1]
    mean = acc * (1.0 / jnp.maximum(cnt, 1.0))
    y = (jnp.dot(mean, wl_ref[...], preferred_element_type=jnp.float32)
         + b_ref[...]
         + jnp.dot(xin, wr_ref[...], preferred_element_type=jnp.float32))
    if relu:
        h = jnp.maximum(y, 0.0)
        out_ref[0] = h[:, :DH]
        out_ref[1] = h[:, DH:]
    else:
        out_ref[...] = y


def _combine(pa, pc, xin, wlT, wrT, b2d, relu):
    R = 10000
    grid = (N // R,)
    if relu:
        out_shape = jax.ShapeDtypeStruct((NC, N, DH), jnp.float32)
        out_spec = pl.BlockSpec((NC, R, DH), lambda i: (0, i, 0))
    else:
        out_shape = jax.ShapeDtypeStruct((N, D), jnp.float32)
        out_spec = pl.BlockSpec((R, D), lambda i: (i, 0))
    return pl.pallas_call(
        functools.partial(_combine_body, relu),
        grid=grid,
        in_specs=[
            pl.BlockSpec((NC, R, DH), lambda i: (0, i, 0)),
            pl.BlockSpec((NC, R, 16), lambda i: (0, i, 0)),
            pl.BlockSpec((NC, R, DH), lambda i: (0, i, 0)),
            pl.BlockSpec((D, D), lambda i: (0, 0)),
            pl.BlockSpec((D, D), lambda i: (0, 0)),
            pl.BlockSpec((1, D), lambda i: (0, 0)),
        ],
        out_specs=out_spec,
        out_shape=out_shape,
    )(pa, pc, xin, wlT, wrT, b2d)


@jax.jit
def kernel(x, edge_index, W1l, b1l, W1r, W2l, b2l, W2r):
    src = edge_index[0]
    dst = edge_index[1]
    pad = EP - E
    src_p = jnp.concatenate([src, jnp.zeros((pad,), jnp.int32)]).reshape(NS, K, C)
    dst_p = jnp.concatenate([dst, jnp.full((pad,), NP - 1, jnp.int32)]).reshape(NS, K, C)
    xs = jnp.stack([x[:, :DH], x[:, DH:]])

    agg1 = _make_sc_agg(True)
    agg2 = _make_sc_agg(False)

    pa1, pc = agg1(xs, src_p, dst_p)
    hs = _combine(pa1, pc, xs, W1l.T, W1r.T, b1l.reshape(1, D), relu=True)
    pa2 = agg2(hs, src_p, dst_p)
    if isinstance(pa2, (tuple, list)):
        pa2 = pa2[0]
    out = _combine(pa2, pc, hs, W2l.T, W2r.T, b2l.reshape(1, D), relu=False)
    return out
